# baseline (speedup 1.0000x reference)
import functools

import jax
import jax.numpy as jnp
from jax import lax
from jax.experimental import pallas as pl
from jax.experimental.pallas import tpu as pltpu
from jax.experimental.pallas import tpu_sc as plsc

B, T = 4096, 200
V, E, H = 1000000, 64, 32

NC, NS = 2, 16
NW = NC * NS
TCH = 40
NCHT = T // TCH
TT = TCH // 8
CROWS = TCH * B
CHUNK = 128
NBUF = 5

B2 = B // 2
E2, H2 = 2 * E, 2 * H
TPB = 8


def _gather_body(tc0, table_hbm, x4_hbm, out_hbm, idx_v, rows_v, *sems):
    wid = lax.axis_index("s") * NC + lax.axis_index("c")

    pltpu.sync_copy(x4_hbm.at[pl.ds(tc0, TT), wid], idx_v)

    def gather_copy(c, slot):
        return pltpu.make_async_copy(
            table_hbm.at[idx_v.at[c // 8, c % 8]],
            rows_v.at[slot], sems[slot])

    for b in range(NBUF):
        gather_copy(b, b).start()

    def body(i, carry):
        for b in range(NBUF):
            c = i * NBUF + b
            gather_copy(c, b).wait()
            pltpu.sync_copy(
                rows_v.at[b, :, pl.ds(0, E)],
                out_hbm.at[pl.ds(c * B + wid * CHUNK, CHUNK)])
            nxt = c + NBUF

            @pl.when(nxt < TCH)
            def _():
                gather_copy(nxt, b).start()
        return carry

    lax.fori_loop(0, TCH // NBUF, body, 0)


@functools.cache
def _make_sc_gather(k):
    return pl.kernel(
        functools.partial(_gather_body, k * TT),
        out_type=jax.ShapeDtypeStruct((CROWS, E), jnp.float32),
        mesh=plsc.VectorSubcoreMesh(core_axis_name="c", subcore_axis_name="s"),
        scratch_types=[
            pltpu.VMEM((TT, 8, CHUNK), jnp.int32),
            pltpu.VMEM((NBUF, CHUNK, 2 * E), jnp.float32),
        ] + [pltpu.SemaphoreType.DMA] * NBUF,
        compiler_params=pltpu.CompilerParams(
            use_tc_tiling_on_sc=False, needs_layout_passes=False),
    )


def _lstm_body(emb_ref, wx_ref, wh_ref, b_ref,
               hin_ref, cin_ref, hout_ref, cout_ref, h_scr, c_scr):
    tt = pl.program_id(0)

    @pl.when(tt == 0)
    def _():
        h_scr[...] = hin_ref[...]
        c_scr[...] = cin_ref[...]

    h = h_scr[...]
    c = c_scr[...]
    for p in range(TPB):
        xt = emb_ref[p]

        def gate(k, h=h, xt=xt):
            return (
                lax.dot_general(xt, wx_ref[k], (((1,), (0,)), ((), ())),
                                preferred_element_type=jnp.float32)
                + lax.dot_general(h, wh_ref[k], (((1,), (0,)), ((), ())),
                                  preferred_element_type=jnp.float32)
                + b_ref[k]
            )

        i = jax.nn.sigmoid(gate(0))
        f = jax.nn.sigmoid(gate(1))
        g = jnp.tanh(gate(2))
        o = jax.nn.sigmoid(gate(3))
        c = f * c + i * g
        h = o * jnp.tanh(c)
    h_scr[...] = h
    c_scr[...] = c

    @pl.when(tt == TCH // TPB - 1)
    def _():
        hout_ref[...] = h
        cout_ref[...] = c


def _lstm_chunk(emb_c, wx2, wh2, bias2, h, c, interpret=False):
    state = pl.BlockSpec((B2, H2), lambda t: (0, 0))
    return pl.pallas_call(
        _lstm_body,
        grid=(TCH // TPB,),
        in_specs=[
            pl.BlockSpec((TPB, B2, E2), lambda t: (t, 0, 0)),
            pl.BlockSpec((4, E2, H2), lambda t: (0, 0, 0)),
            pl.BlockSpec((4, H2, H2), lambda t: (0, 0, 0)),
            pl.BlockSpec((4, 1, H2), lambda t: (0, 0, 0)),
            state, state,
        ],
        out_specs=(state, state),
        out_shape=(jax.ShapeDtypeStruct((B2, H2), jnp.float32),
                   jax.ShapeDtypeStruct((B2, H2), jnp.float32)),
        scratch_shapes=[
            pltpu.VMEM((B2, H2), jnp.float32),
            pltpu.VMEM((B2, H2), jnp.float32),
        ],
        compiler_params=pltpu.CompilerParams(
            dimension_semantics=("arbitrary",)),
        interpret=interpret,
    )(emb_c, wx2, wh2, bias2, h, c)


def _blockdiag(w):
    k4, K, Hh = w.shape
    z = jnp.zeros((k4, K, Hh), w.dtype)
    top = jnp.concatenate([w, z], axis=2)
    bot = jnp.concatenate([z, w], axis=2)
    return jnp.concatenate([top, bot], axis=1)


def kernel(x, emb, W_ih, W_hh, b_ih, b_hh):
    x4 = jnp.transpose(
        x.astype(jnp.int32).reshape(NW, CHUNK, T // 8, 8), (2, 0, 3, 1))
    wx = jnp.transpose(W_ih.reshape(4, H, E), (0, 2, 1))
    wh = jnp.transpose(W_hh.reshape(4, H, H), (0, 2, 1))
    bias = (b_ih + b_hh).reshape(4, 1, H)
    wx2, wh2 = _blockdiag(wx), _blockdiag(wh)
    bias2 = jnp.concatenate([bias, bias], axis=2)

    emb128 = jnp.concatenate([emb, jnp.zeros((V, E), jnp.float32)], axis=1)

    h = jnp.zeros((B2, H2), jnp.float32)
    c = jnp.zeros((B2, H2), jnp.float32)
    for k in range(NCHT):
        emb_c = _make_sc_gather(k)(emb128, x4).reshape(TCH, B2, E2)
        h, c = _lstm_chunk(emb_c, wx2, wh2, bias2, h, c)
    return h.reshape(B, H)


# speedup vs baseline: 1.0073x; 1.0073x over previous
import functools

import jax
import jax.numpy as jnp
from jax import lax
from jax.experimental import pallas as pl
from jax.experimental.pallas import tpu as pltpu
from jax.experimental.pallas import tpu_sc as plsc

B, T = 4096, 200
V, E, H = 1000000, 64, 32

NC, NS = 2, 16
NW = NC * NS
TCH = 40
NCHT = T // TCH
TT = TCH // 8
CROWS = TCH * B
CHUNK = 128
NBUF = 5

B2 = B // 2
E2, H2 = 2 * E, 2 * H
TPB = 4


def _gather_body(tc0, table_hbm, x4_hbm, out_hbm, idx_v, rows_v, *sems):
    wid = lax.axis_index("s") * NC + lax.axis_index("c")

    pltpu.sync_copy(x4_hbm.at[pl.ds(tc0, TT), wid], idx_v)

    def gather_copy(c, slot):
        return pltpu.make_async_copy(
            table_hbm.at[idx_v.at[c // 8, c % 8]],
            rows_v.at[slot], sems[slot])

    for b in range(NBUF):
        gather_copy(b, b).start()

    def body(i, carry):
        for b in range(NBUF):
            c = i * NBUF + b
            gather_copy(c, b).wait()
            pltpu.sync_copy(
                rows_v.at[b, :, pl.ds(0, E)],
                out_hbm.at[pl.ds(c * B + wid * CHUNK, CHUNK)])
            nxt = c + NBUF

            @pl.when(nxt < TCH)
            def _():
                gather_copy(nxt, b).start()
        return carry

    lax.fori_loop(0, TCH // NBUF, body, 0)


@functools.cache
def _make_sc_gather(k):
    return pl.kernel(
        functools.partial(_gather_body, k * TT),
        out_type=jax.ShapeDtypeStruct((CROWS, E), jnp.float32),
        mesh=plsc.VectorSubcoreMesh(core_axis_name="c", subcore_axis_name="s"),
        scratch_types=[
            pltpu.VMEM((TT, 8, CHUNK), jnp.int32),
            pltpu.VMEM((NBUF, CHUNK, 2 * E), jnp.float32),
        ] + [pltpu.SemaphoreType.DMA] * NBUF,
        compiler_params=pltpu.CompilerParams(
            use_tc_tiling_on_sc=False, needs_layout_passes=False),
    )


def _lstm_body(emb_ref, wx_ref, wh_ref, b_ref,
               hin_ref, cin_ref, hout_ref, cout_ref, h_scr, c_scr):
    tt = pl.program_id(0)

    @pl.when(tt == 0)
    def _():
        h_scr[...] = hin_ref[...]
        c_scr[...] = cin_ref[...]

    h = h_scr[...]
    c = c_scr[...]
    for p in range(TPB):
        xt = emb_ref[p]

        def gate(k, h=h, xt=xt):
            return (
                lax.dot_general(xt, wx_ref[k], (((1,), (0,)), ((), ())),
                                preferred_element_type=jnp.float32)
                + lax.dot_general(h, wh_ref[k], (((1,), (0,)), ((), ())),
                                  preferred_element_type=jnp.float32)
                + b_ref[k]
            )

        i = jax.nn.sigmoid(gate(0))
        f = jax.nn.sigmoid(gate(1))
        g = jnp.tanh(gate(2))
        o = jax.nn.sigmoid(gate(3))
        c = f * c + i * g
        h = o * jnp.tanh(c)
    h_scr[...] = h
    c_scr[...] = c

    @pl.when(tt == TCH // TPB - 1)
    def _():
        hout_ref[...] = h
        cout_ref[...] = c


def _lstm_chunk(emb_c, wx2, wh2, bias2, h, c, interpret=False):
    state = pl.BlockSpec((B2, H2), lambda t: (0, 0))
    return pl.pallas_call(
        _lstm_body,
        grid=(TCH // TPB,),
        in_specs=[
            pl.BlockSpec((TPB, B2, E2), lambda t: (t, 0, 0)),
            pl.BlockSpec((4, E2, H2), lambda t: (0, 0, 0)),
            pl.BlockSpec((4, H2, H2), lambda t: (0, 0, 0)),
            pl.BlockSpec((4, 1, H2), lambda t: (0, 0, 0)),
            state, state,
        ],
        out_specs=(state, state),
        out_shape=(jax.ShapeDtypeStruct((B2, H2), jnp.float32),
                   jax.ShapeDtypeStruct((B2, H2), jnp.float32)),
        scratch_shapes=[
            pltpu.VMEM((B2, H2), jnp.float32),
            pltpu.VMEM((B2, H2), jnp.float32),
        ],
        compiler_params=pltpu.CompilerParams(
            dimension_semantics=("arbitrary",)),
        interpret=interpret,
    )(emb_c, wx2, wh2, bias2, h, c)


def _blockdiag(w):
    k4, K, Hh = w.shape
    z = jnp.zeros((k4, K, Hh), w.dtype)
    top = jnp.concatenate([w, z], axis=2)
    bot = jnp.concatenate([z, w], axis=2)
    return jnp.concatenate([top, bot], axis=1)


def kernel(x, emb, W_ih, W_hh, b_ih, b_hh):
    x4 = jnp.transpose(
        x.astype(jnp.int32).reshape(NW, CHUNK, T // 8, 8), (2, 0, 3, 1))
    wx = jnp.transpose(W_ih.reshape(4, H, E), (0, 2, 1))
    wh = jnp.transpose(W_hh.reshape(4, H, H), (0, 2, 1))
    bias = (b_ih + b_hh).reshape(4, 1, H)
    wx2, wh2 = _blockdiag(wx), _blockdiag(wh)
    bias2 = jnp.concatenate([bias, bias], axis=2)

    emb128 = jnp.concatenate([emb, jnp.zeros((V, E), jnp.float32)], axis=1)

    h = jnp.zeros((B2, H2), jnp.float32)
    c = jnp.zeros((B2, H2), jnp.float32)
    for k in range(NCHT):
        emb_c = _make_sc_gather(k)(emb128, x4).reshape(TCH, B2, E2)
        h, c = _lstm_chunk(emb_c, wx2, wh2, bias2, h, c)
    return h.reshape(B, H)
